# baseline (speedup 1.0000x reference)
import functools
import math

import jax
import jax.numpy as jnp
from jax import lax
from jax.experimental import pallas as pl
from jax.experimental.pallas import tpu as pltpu
from jax.experimental.pallas import tpu_sc as plsc

D_MODEL = 768
LANES = 16
NUM_CORES = 2
NUM_SUBCORES = 16
NUM_WORKERS = NUM_CORES * NUM_SUBCORES
SCALE = math.sqrt(D_MODEL)


def _make_sc_gather(batch: int, seq_len: int):
    n_rows = batch * seq_len
    rows_per_worker = n_rows // NUM_WORKERS
    workers_per_batch_row = seq_len // rows_per_worker
    chunk = 32
    nbuf = 4
    n_chunks = rows_per_worker // chunk
    n_groups = n_chunks // nbuf
    vecs_per_row = D_MODEL // LANES

    mesh = plsc.VectorSubcoreMesh(core_axis_name="c", subcore_axis_name="s")

    @functools.partial(
        pl.kernel,
        out_type=jax.ShapeDtypeStruct((n_rows, D_MODEL), jnp.float32),
        mesh=mesh,
        scratch_types=[
            pltpu.VMEM((rows_per_worker,), jnp.int32),
            pltpu.VMEM((nbuf, chunk, D_MODEL), jnp.float32),
            [pltpu.SemaphoreType.DMA] * nbuf,
            [pltpu.SemaphoreType.DMA] * nbuf,
        ],
    )
    def sc_gather(table_hbm, idx_hbm, out_hbm, idx_v, buf, sems_in, sems_out):
        wid = lax.axis_index("s") * NUM_CORES + lax.axis_index("c")
        base = wid * rows_per_worker
        brow = wid // workers_per_batch_row
        bcol = (wid % workers_per_batch_row) * rows_per_worker
        pltpu.sync_copy(idx_hbm.at[brow, pl.ds(bcol, rows_per_worker)], idx_v)

        def gather_desc(b, ch):
            return pltpu.make_async_copy(
                table_hbm.at[idx_v.at[pl.ds(ch * chunk, chunk)]],
                buf.at[b],
                sems_in[b],
            )

        def store_desc(b, ch):
            return pltpu.make_async_copy(
                buf.at[b],
                out_hbm.at[pl.ds(base + ch * chunk, chunk)],
                sems_out[b],
            )

        gather_desc(0, 0).start()
        gather_desc(1, 1).start()

        def group_body(g, _):
            for u in range(nbuf):
                b = u
                ch = g * nbuf + u

                gather_desc(b, ch).wait()

                def scale_row(r, b=b):
                    for k in range(vecs_per_row):
                        sl = pl.ds(k * LANES, LANES)
                        buf[b, r, sl] = buf[b, r, sl] * SCALE

                plsc.parallel_loop(0, chunk, unroll=2)(scale_row)

                pb = (u - 2) % nbuf
                if u >= 2:
                    store_desc(pb, ch - 2).wait()

                    @pl.when(g < n_groups - 1)
                    def _refill():
                        gather_desc(pb, ch + 2).start()
                else:
                    @pl.when(g > 0)
                    def _drain():
                        store_desc(pb, ch - 2).wait()

                    gather_desc(pb, ch + 2).start()

                store_desc(b, ch).start()
            return _

        lax.fori_loop(0, n_groups, group_body, None)
        store_desc((n_chunks - 2) % nbuf, n_chunks - 2).wait()
        store_desc((n_chunks - 1) % nbuf, n_chunks - 1).wait()

    return sc_gather


def kernel(inputs, table):
    batch, seq_len = inputs.shape
    gathered = _make_sc_gather(batch, seq_len)(table, inputs)
    return gathered.reshape(batch, seq_len, D_MODEL)


# speedup vs baseline: 1.0431x; 1.0431x over previous
import functools
import math

import jax
import jax.numpy as jnp
from jax import lax
from jax.experimental import pallas as pl
from jax.experimental.pallas import tpu as pltpu
from jax.experimental.pallas import tpu_sc as plsc

D_MODEL = 768
LANES = 16
NUM_CORES = 2
NUM_SUBCORES = 16
NUM_WORKERS = NUM_CORES * NUM_SUBCORES
SCALE = math.sqrt(D_MODEL)


def _make_sc_gather(batch: int, seq_len: int):
    n_rows = batch * seq_len
    rows_per_worker = n_rows // NUM_WORKERS
    workers_per_batch_row = seq_len // rows_per_worker
    chunk = 32
    nbuf = 4
    n_chunks = rows_per_worker // chunk
    n_groups = n_chunks // nbuf
    vecs_per_row = D_MODEL // LANES

    mesh = plsc.VectorSubcoreMesh(core_axis_name="c", subcore_axis_name="s")

    @functools.partial(
        pl.kernel,
        out_type=jax.ShapeDtypeStruct((n_rows, D_MODEL), jnp.float32),
        mesh=mesh,
        scratch_types=[
            pltpu.VMEM((rows_per_worker,), jnp.int32),
            pltpu.VMEM((nbuf, chunk, D_MODEL), jnp.float32),
            [pltpu.SemaphoreType.DMA] * nbuf,
            [pltpu.SemaphoreType.DMA] * nbuf,
        ],
    )
    def sc_gather(table_hbm, idx_hbm, out_hbm, idx_v, buf, sems_in, sems_out):
        wid = lax.axis_index("s") * NUM_CORES + lax.axis_index("c")
        base = wid * rows_per_worker
        brow = wid // workers_per_batch_row
        bcol = (wid % workers_per_batch_row) * rows_per_worker
        pltpu.sync_copy(idx_hbm.at[brow, pl.ds(bcol, rows_per_worker)], idx_v)

        def gather_desc(b, ch):
            return pltpu.make_async_copy(
                table_hbm.at[idx_v.at[pl.ds(ch * chunk, chunk)]],
                buf.at[b],
                sems_in[b],
            )

        def store_desc(b, ch):
            return pltpu.make_async_copy(
                buf.at[b],
                out_hbm.at[pl.ds(base + ch * chunk, chunk)],
                sems_out[b],
            )

        gather_desc(0, 0).start()
        gather_desc(1, 1).start()

        def group_body(g, _):
            for u in range(nbuf):
                b = u
                ch = g * nbuf + u

                gather_desc(b, ch).wait()

                pb = (u - 2) % nbuf
                if u >= 2:
                    store_desc(pb, ch - 2).wait()

                    @pl.when(g < n_groups - 1)
                    def _refill():
                        gather_desc(pb, ch + 2).start()
                else:
                    @pl.when(g > 0)
                    def _drain():
                        store_desc(pb, ch - 2).wait()

                    gather_desc(pb, ch + 2).start()

                def scale_row(r, b=b):
                    for k in range(vecs_per_row):
                        sl = pl.ds(k * LANES, LANES)
                        buf[b, r, sl] = buf[b, r, sl] * SCALE

                plsc.parallel_loop(0, chunk)(scale_row)

                store_desc(b, ch).start()
            return _

        lax.fori_loop(0, n_groups, group_body, None)
        store_desc((n_chunks - 2) % nbuf, n_chunks - 2).wait()
        store_desc((n_chunks - 1) % nbuf, n_chunks - 1).wait()

    return sc_gather


def kernel(inputs, table):
    batch, seq_len = inputs.shape
    gathered = _make_sc_gather(batch, seq_len)(table, inputs)
    return gathered.reshape(batch, seq_len, D_MODEL)
